# baseline (speedup 1.0000x reference)
import functools

import jax
import jax.numpy as jnp
from jax import lax
from jax.experimental import pallas as pl
from jax.experimental.pallas import tpu as pltpu
from jax.experimental.pallas import tpu_sc as plsc

S = 819200
D = 64
DPAD = 128
VOCAB = 1000000

NC = 2
NS = 16
NW = NC * NS
B_W = S // NW
C = 128
N_CHUNK = B_W // C
N_PAIR = N_CHUNK // 2

_mesh = plsc.VectorSubcoreMesh(core_axis_name="c", subcore_axis_name="s")


@functools.partial(
    pl.kernel,
    mesh=_mesh,
    out_type=jax.ShapeDtypeStruct((D, S), jnp.float32),
    scratch_types=[
        pltpu.VMEM((B_W,), jnp.int32),
        pltpu.VMEM((C, DPAD), jnp.float32),
        pltpu.VMEM((C, DPAD), jnp.float32),
        pltpu.VMEM((D, C), jnp.float32),
        pltpu.VMEM((D, C), jnp.float32),
        pltpu.SemaphoreType.DMA,
        pltpu.SemaphoreType.DMA,
        pltpu.SemaphoreType.DMA,
        pltpu.SemaphoreType.DMA,
    ],
    compiler_params=pltpu.CompilerParams(
        use_tc_tiling_on_sc=True, needs_layout_passes=False),
)
def _sc_gather_t(table, idx, out_t, idx_v, rows0, rows1, buft0, buft1,
                 gsem0, gsem1, wsem0, wsem1):
    wid = lax.axis_index("s") * NC + lax.axis_index("c")
    base = wid * B_W
    pltpu.sync_copy(idx.at[pl.ds(base, B_W)], idx_v)

    iota16 = lax.iota(jnp.int32, 16)

    def gather_cp(chunk, buf, sem):
        return pltpu.make_async_copy(
            table.at[idx_v.at[pl.ds(chunk * C, C)]], buf, sem)

    def transpose_chunk(rows, buft):
        def kbody(k, carry):
            idx_i = k * 16 + iota16
            for c0 in range(0, D, 8):
                vecs = [
                    plsc.load_gather(
                        rows, [idx_i, jnp.full((16,), c, jnp.int32)])
                    for c in range(c0, c0 + 8)
                ]
                for c, vec in zip(range(c0, c0 + 8), vecs):
                    buft[c, pl.ds(k * 16, 16)] = vec
            return carry
        lax.fori_loop(0, C // 16, kbody, 0)

    def write_cps(chunk, buft, sem):
        col = base + chunk * C
        return [
            pltpu.make_async_copy(
                buft.at[pl.ds(8 * g, 8), :],
                out_t.at[pl.ds(8 * g, 8), pl.ds(col, C)],
                sem,
            )
            for g in range(8)
        ]

    gather_cp(0, rows0, gsem0).start()

    def body(gg, carry):
        e = 2 * gg
        o = e + 1
        gather_cp(e, rows0, gsem0).wait()
        gather_cp(o, rows1, gsem1).start()
        transpose_chunk(rows0, buft0)
        for cp in write_cps(e, buft0, wsem0):
            cp.start()
        gather_cp(o, rows1, gsem1).wait()

        @pl.when(o + 1 < N_CHUNK)
        def _():
            gather_cp(o + 1, rows0, gsem0).start()

        transpose_chunk(rows1, buft1)
        for cp in write_cps(e, buft0, wsem0):
            cp.wait()
        for cp in write_cps(o, buft1, wsem1):
            cp.start()
        for cp in write_cps(o, buft1, wsem1):
            cp.wait()
        return carry

    lax.fori_loop(0, N_PAIR, body, 0)


def kernel(flat_source, flat_idx):
    padded = jnp.pad(flat_source, ((0, 0), (0, DPAD - D)))
    out_t = _sc_gather_t(padded, flat_idx.astype(jnp.int32))
    return out_t.T


# speedup vs baseline: 1.2883x; 1.2883x over previous
import functools

import jax
import jax.numpy as jnp
from jax import lax
from jax.experimental import pallas as pl
from jax.experimental.pallas import tpu as pltpu
from jax.experimental.pallas import tpu_sc as plsc

S = 819200
D = 64
DPAD = 128

NC = 2
NS = 16
NW = NC * NS
B_W = S // NW
C = 256
N_CHUNK = B_W // C
N_PAIR = N_CHUNK // 2

_mesh = plsc.VectorSubcoreMesh(core_axis_name="c", subcore_axis_name="s")


@functools.partial(
    pl.kernel,
    mesh=_mesh,
    out_type=jax.ShapeDtypeStruct((S, DPAD), jnp.float32),
    scratch_types=[
        pltpu.VMEM((B_W,), jnp.int32),
        pltpu.VMEM((C, DPAD), jnp.float32),
        pltpu.VMEM((C, DPAD), jnp.float32),
        pltpu.SemaphoreType.DMA,
        pltpu.SemaphoreType.DMA,
        pltpu.SemaphoreType.DMA,
        pltpu.SemaphoreType.DMA,
    ],
    compiler_params=pltpu.CompilerParams(
        use_tc_tiling_on_sc=True, needs_layout_passes=False),
)
def _sc_gather128(table, idx, out, idx_v, rows0, rows1,
                  gsem0, gsem1, wsem0, wsem1):
    wid = lax.axis_index("s") * NC + lax.axis_index("c")
    base = wid * B_W
    pltpu.sync_copy(idx.at[pl.ds(base, B_W)], idx_v)

    def gather_cp(chunk, buf, sem):
        return pltpu.make_async_copy(
            table.at[idx_v.at[pl.ds(chunk * C, C)]], buf, sem)

    def write_cp(chunk, buf, sem):
        return pltpu.make_async_copy(
            buf, out.at[pl.ds(base + chunk * C, C)], sem)

    gather_cp(0, rows0, gsem0).start()

    def body(gg, carry):
        e = 2 * gg
        o = e + 1
        gather_cp(e, rows0, gsem0).wait()
        gather_cp(o, rows1, gsem1).start()
        write_cp(e, rows0, wsem0).start()
        gather_cp(o, rows1, gsem1).wait()
        write_cp(e, rows0, wsem0).wait()

        @pl.when(o + 1 < N_CHUNK)
        def _():
            gather_cp(o + 1, rows0, gsem0).start()

        write_cp(o, rows1, wsem1).start()
        write_cp(o, rows1, wsem1).wait()
        return carry

    lax.fori_loop(0, N_PAIR, body, 0)


def kernel(flat_source, flat_idx):
    padded = jnp.pad(flat_source, ((0, 0), (0, DPAD - D)))
    out128 = _sc_gather128(padded, flat_idx.astype(jnp.int32))
    return out128[:, :D]
